# baseline (speedup 1.0000x reference)
import functools

import jax
import jax.numpy as jnp
from jax import lax
from jax.experimental import pallas as pl
from jax.experimental.pallas import tpu as pltpu
from jax.experimental.pallas import tpu_sc as plsc

K_NN = 9
BIG = 1e30
SEG_W = 128
TILE = 2048
QBLK = 128


def _phase1_kernel(patch_ref, bank_ref, dist_ref, segmin_ref, *, n_rows,
                   tile):
    i = pl.program_id(0)
    q = patch_ref[...]
    b = bank_ref[...]
    ones = jnp.ones((1, b.shape[1]), jnp.float32)
    b2 = lax.dot_general(ones, b * b, (((1,), (1,)), ((), ())),
                         preferred_element_type=jnp.float32)
    q2 = jnp.sum(q * q, axis=1, keepdims=True)
    dot = lax.dot_general(q, b, (((1,), (1,)), ((), ())),
                          preferred_element_type=jnp.float32)
    d2 = q2 + (b2 - 2.0 * dot)
    dist = jnp.sqrt(jnp.maximum(d2, 1e-12))
    n_full = n_rows // tile

    @pl.when(i >= n_full)
    def _mask():
        col = lax.broadcasted_iota(jnp.int32, dist.shape, 1) + i * tile
        dist_ref[...] = jnp.where(col < n_rows, dist, BIG)

    @pl.when(i < n_full)
    def _store():
        dist_ref[...] = dist

    d = dist_ref[...]
    segs = [jnp.min(d[:, g * SEG_W:(g + 1) * SEG_W], axis=1, keepdims=True)
            for g in range(tile // SEG_W)]
    segmin_ref[...] = jnp.concatenate(segs, axis=1)[None]


def _phase2_kernel(dist_flat, segmin_flat, out_hbm, seg_v, gath_v, out_v,
                   r2_v, sem, *, n_seg, n_q):
    seg_c = n_seg // 16
    wid = lax.axis_index("s") * 2 + lax.axis_index("c")
    q0 = wid * 16
    iota16 = lax.iota(jnp.int32, 16)

    seg_copies = [
        pltpu.async_copy(
            segmin_flat.at[pl.ds((c * n_q + q0) * 16, 256)],
            seg_v.at[pl.ds(c * 256, 256)], sem)
        for c in range(seg_c)
    ]
    for cp in seg_copies:
        cp.wait()

    def one_query(qi, _):
        rk = seg_v[pl.ds(qi * 16, 16)]
        rv = iota16
        rk, rv = plsc.sort_key_val(rk, rv)

        def merge_seg(c, carry):
            rk, rv = carry
            k = seg_v[pl.ds(c * 256 + qi * 16, 16)]
            v = iota16 + c * 16
            kd, vd = plsc.sort_key_val(k, v, descending=True)
            mk = jnp.minimum(rk, kd)
            mv = jnp.where(kd < rk, vd, rv)
            return tuple(plsc.sort_key_val(mk, mv))

        rk, rv = lax.fori_loop(1, seg_c, merge_seg, (rk, rv))
        thr = jnp.max(jnp.where(iota16 == 8, rk, -BIG), axis=0)

        copies = []
        for j in range(K_NN):
            seg_j = jnp.max(jnp.where(iota16 == j, rv, -1), axis=0)
            start = ((q0 + qi) * n_seg + seg_j) * SEG_W
            copies.append(pltpu.async_copy(
                dist_flat.at[pl.ds(start, SEG_W)],
                gath_v.at[pl.ds(j * SEG_W, SEG_W)], sem))
        for c in copies:
            c.wait()

        r2_v[...] = jnp.full((16,), BIG, jnp.float32)
        for j in range(K_NN):
            for w in range(SEG_W // 16):
                k = gath_v[pl.ds(j * SEG_W + w * 16, 16)]
                cnt = jnp.sum((k <= thr).astype(jnp.int32), axis=0)

                @pl.when(cnt > 0)
                def _merge():
                    kd, _ = plsc.sort_key_val(k, iota16, descending=True)
                    mk = jnp.minimum(r2_v[...], kd)
                    ks, _ = plsc.sort_key_val(mk, iota16)
                    r2_v[...] = ks

        out_v[pl.ds(qi * 16, 16)] = r2_v[...]
        return 0

    lax.fori_loop(0, 16, one_query, 0)
    pltpu.sync_copy(out_v, out_hbm.at[pl.ds(q0 * 16, 256)])


@jax.jit
def kernel(patch, memory_bank):
    q_n, d = patch.shape
    n_rows = memory_bank.shape[0]
    n_tiles = pl.cdiv(n_rows, TILE)
    n_cols = n_tiles * TILE
    n_seg = n_cols // SEG_W
    n_qblk = q_n // QBLK

    dist, segmin = pl.pallas_call(
        functools.partial(_phase1_kernel, n_rows=n_rows, tile=TILE),
        grid=(n_tiles, n_qblk),
        in_specs=[
            pl.BlockSpec((QBLK, d), lambda i, j: (j, 0)),
            pl.BlockSpec((TILE, d), lambda i, j: (i, 0)),
        ],
        out_specs=[
            pl.BlockSpec((QBLK, TILE), lambda i, j: (j, i)),
            pl.BlockSpec((1, QBLK, TILE // SEG_W), lambda i, j: (i, j, 0)),
        ],
        out_shape=[
            jax.ShapeDtypeStruct((q_n, n_cols), jnp.float32),
            jax.ShapeDtypeStruct((n_tiles, q_n, TILE // SEG_W), jnp.float32),
        ],
        compiler_params=pltpu.CompilerParams(
            dimension_semantics=("arbitrary", "arbitrary")),
    )(patch, memory_bank)

    dist_flat = dist.reshape(-1)
    segmin_flat = segmin.reshape(-1)

    mesh = plsc.VectorSubcoreMesh(core_axis_name="c", subcore_axis_name="s")
    out = pl.kernel(
        functools.partial(_phase2_kernel, n_seg=n_seg, n_q=q_n),
        mesh=mesh,
        out_type=jax.ShapeDtypeStruct((q_n * 16,), jnp.float32),
        scratch_types=[
            pltpu.VMEM((16 * n_seg,), jnp.float32),
            pltpu.VMEM((K_NN * SEG_W,), jnp.float32),
            pltpu.VMEM((16 * 16,), jnp.float32),
            pltpu.VMEM((16,), jnp.float32),
            pltpu.SemaphoreType.DMA,
        ],
        compiler_params=pltpu.CompilerParams(needs_layout_passes=False),
    )(dist_flat, segmin_flat)
    return out.reshape(q_n, 16)[:, :K_NN]


# speedup vs baseline: 1.4526x; 1.4526x over previous
import functools

import jax
import jax.numpy as jnp
from jax import lax
from jax.experimental import pallas as pl
from jax.experimental.pallas import tpu as pltpu
from jax.experimental.pallas import tpu_sc as plsc

K_NN = 9
BIG = 1e30
SEG_W = 128
TILE = 2048
QBLK = 128


def _phase1_kernel(patch_ref, bank_ref, dist_ref, segmin_ref, *, n_rows,
                   tile):
    i = pl.program_id(0)
    q = patch_ref[...]
    b = bank_ref[...]
    ones = jnp.ones((1, b.shape[1]), jnp.float32)
    b2 = lax.dot_general(ones, b * b, (((1,), (1,)), ((), ())),
                         preferred_element_type=jnp.float32)
    q2 = jnp.sum(q * q, axis=1, keepdims=True)
    dot = lax.dot_general(q, b, (((1,), (1,)), ((), ())),
                          preferred_element_type=jnp.float32)
    d2 = q2 + (b2 - 2.0 * dot)
    dist = jnp.sqrt(jnp.maximum(d2, 1e-12))
    n_full = n_rows // tile

    @pl.when(i >= n_full)
    def _mask():
        col = lax.broadcasted_iota(jnp.int32, dist.shape, 1) + i * tile
        dist_ref[...] = jnp.where(col < n_rows, dist, BIG)

    @pl.when(i < n_full)
    def _store():
        dist_ref[...] = dist

    d = dist_ref[...]
    segs = [jnp.min(d[:, g * SEG_W:(g + 1) * SEG_W], axis=1, keepdims=True)
            for g in range(tile // SEG_W)]
    segmin_ref[...] = jnp.concatenate(segs, axis=1)[None]


def _phase2_kernel(dist_flat, segmin_flat, out_hbm, seg_v, gath_v, out_v,
                   r2_v, sem, *, n_seg, n_q):
    seg_c = n_seg // 16
    wid = lax.axis_index("s") * 2 + lax.axis_index("c")
    q0 = wid * 16
    iota16 = lax.iota(jnp.int32, 16)

    seg_copies = [
        pltpu.async_copy(
            segmin_flat.at[pl.ds((c * n_q + q0) * 16, 256)],
            seg_v.at[pl.ds(c * 256, 256)], sem)
        for c in range(seg_c)
    ]
    for cp in seg_copies:
        cp.wait()

    def one_query(qi, _):
        rk = seg_v[pl.ds(qi * 16, 16)]
        rv = iota16
        rk, rv = plsc.sort_key_val(rk, rv)

        def merge_seg(c, carry):
            rk, rv = carry
            k = seg_v[pl.ds(c * 256 + qi * 16, 16)]
            v = iota16 + c * 16
            kd, vd = plsc.sort_key_val(k, v, descending=True)
            mk = jnp.minimum(rk, kd)
            mv = jnp.where(kd < rk, vd, rv)
            return tuple(plsc.sort_key_val(mk, mv))

        rk, rv = lax.fori_loop(1, seg_c, merge_seg, (rk, rv))
        thr = jnp.max(jnp.where(iota16 == 8, rk, -BIG), axis=0)

        copies = []
        for j in range(K_NN):
            seg_j = jnp.max(jnp.where(iota16 == j, rv, -1), axis=0)
            copies.append(pltpu.async_copy(
                dist_flat.at[q0 + qi, pl.ds(seg_j * SEG_W, SEG_W)],
                gath_v.at[pl.ds(j * SEG_W, SEG_W)], sem))
        for c in copies:
            c.wait()

        r2_v[...] = jnp.full((16,), BIG, jnp.float32)
        for j in range(K_NN):
            for w in range(SEG_W // 16):
                k = gath_v[pl.ds(j * SEG_W + w * 16, 16)]
                cnt = jnp.sum((k <= thr).astype(jnp.int32), axis=0)

                @pl.when(cnt > 0)
                def _merge():
                    kd, _ = plsc.sort_key_val(k, iota16, descending=True)
                    mk = jnp.minimum(r2_v[...], kd)
                    ks, _ = plsc.sort_key_val(mk, iota16)
                    r2_v[...] = ks

        out_v[pl.ds(qi * 16, 16)] = r2_v[...]
        return 0

    lax.fori_loop(0, 16, one_query, 0)
    pltpu.sync_copy(out_v, out_hbm.at[pl.ds(q0 * 16, 256)])


@jax.jit
def kernel(patch, memory_bank):
    q_n, d = patch.shape
    n_rows = memory_bank.shape[0]
    n_tiles = pl.cdiv(n_rows, TILE)
    n_cols = n_tiles * TILE
    n_seg = n_cols // SEG_W
    n_qblk = q_n // QBLK

    dist, segmin = pl.pallas_call(
        functools.partial(_phase1_kernel, n_rows=n_rows, tile=TILE),
        grid=(n_tiles, n_qblk),
        in_specs=[
            pl.BlockSpec((QBLK, d), lambda i, j: (j, 0)),
            pl.BlockSpec((TILE, d), lambda i, j: (i, 0)),
        ],
        out_specs=[
            pl.BlockSpec((QBLK, TILE), lambda i, j: (j, i)),
            pl.BlockSpec((1, QBLK, TILE // SEG_W), lambda i, j: (i, j, 0)),
        ],
        out_shape=[
            jax.ShapeDtypeStruct((q_n, n_cols), jnp.float32),
            jax.ShapeDtypeStruct((n_tiles, q_n, TILE // SEG_W), jnp.float32),
        ],
        compiler_params=pltpu.CompilerParams(
            dimension_semantics=("arbitrary", "arbitrary")),
    )(patch, memory_bank)

    segmin_flat = segmin.reshape(-1)

    mesh = plsc.VectorSubcoreMesh(core_axis_name="c", subcore_axis_name="s")
    out = pl.kernel(
        functools.partial(_phase2_kernel, n_seg=n_seg, n_q=q_n),
        mesh=mesh,
        out_type=jax.ShapeDtypeStruct((q_n * 16,), jnp.float32),
        scratch_types=[
            pltpu.VMEM((16 * n_seg,), jnp.float32),
            pltpu.VMEM((K_NN * SEG_W,), jnp.float32),
            pltpu.VMEM((16 * 16,), jnp.float32),
            pltpu.VMEM((16,), jnp.float32),
            pltpu.SemaphoreType.DMA,
        ],
        compiler_params=pltpu.CompilerParams(needs_layout_passes=False),
    )(dist, segmin_flat)
    return out.reshape(q_n, 16)[:, :K_NN]


# speedup vs baseline: 1.5044x; 1.0356x over previous
import functools

import jax
import jax.numpy as jnp
from jax import lax
from jax.experimental import pallas as pl
from jax.experimental.pallas import tpu as pltpu
from jax.experimental.pallas import tpu_sc as plsc

K_NN = 9
BIG = 1e30
SEG_W = 128
TILE = 2048
QBLK = 128


def _phase1_kernel(patch_ref, bank_ref, dist_ref, segmin_ref, b2_ref, *,
                   n_rows, tile):
    i = pl.program_id(0)
    j = pl.program_id(1)
    q = patch_ref[...]
    b = bank_ref[...]

    @pl.when(j == 0)
    def _b2():
        ones = jnp.ones((1, b.shape[1]), jnp.float32)
        b2_ref[...] = lax.dot_general(ones, b * b, (((1,), (1,)), ((), ())),
                                      preferred_element_type=jnp.float32)

    b2 = b2_ref[...]
    q2 = jnp.sum(q * q, axis=1, keepdims=True)
    dot = lax.dot_general(q, b, (((1,), (1,)), ((), ())),
                          preferred_element_type=jnp.float32)
    d2 = q2 + (b2 - 2.0 * dot)
    n_full = n_rows // tile

    @pl.when(i >= n_full)
    def _mask():
        col = lax.broadcasted_iota(jnp.int32, d2.shape, 1) + i * tile
        dist_ref[...] = jnp.where(col < n_rows, d2, BIG)

    @pl.when(i < n_full)
    def _store():
        dist_ref[...] = d2

    d = dist_ref[...]
    segs = [jnp.min(d[:, g * SEG_W:(g + 1) * SEG_W], axis=1, keepdims=True)
            for g in range(tile // SEG_W)]
    segmin_ref[...] = jnp.concatenate(segs, axis=1)[None]


def _phase2_kernel(dist_flat, segmin_flat, out_hbm, seg_v, gath_v, out_v,
                   r2_v, sem, *, n_seg, n_q):
    seg_c = n_seg // 16
    wid = lax.axis_index("s") * 2 + lax.axis_index("c")
    q0 = wid * 16
    iota16 = lax.iota(jnp.int32, 16)

    seg_copies = [
        pltpu.async_copy(
            segmin_flat.at[pl.ds((c * n_q + q0) * 16, 256)],
            seg_v.at[pl.ds(c * 256, 256)], sem)
        for c in range(seg_c)
    ]
    for cp in seg_copies:
        cp.wait()

    def one_query(qi, _):
        rk = seg_v[pl.ds(qi * 16, 16)]
        rv = iota16
        rk, rv = plsc.sort_key_val(rk, rv)

        def merge_seg(c, carry):
            rk, rv = carry
            k = seg_v[pl.ds(c * 256 + qi * 16, 16)]
            v = iota16 + c * 16
            kd, vd = plsc.sort_key_val(k, v, descending=True)
            mk = jnp.minimum(rk, kd)
            mv = jnp.where(kd < rk, vd, rv)
            return tuple(plsc.sort_key_val(mk, mv))

        rk, rv = lax.fori_loop(1, seg_c, merge_seg, (rk, rv))
        thr = jnp.max(jnp.where(iota16 == 8, rk, -BIG), axis=0)

        copies = []
        for j in range(K_NN):
            seg_j = jnp.max(jnp.where(iota16 == j, rv, -1), axis=0)
            copies.append(pltpu.async_copy(
                dist_flat.at[q0 + qi, pl.ds(seg_j * SEG_W, SEG_W)],
                gath_v.at[pl.ds(j * SEG_W, SEG_W)], sem))
        for c in copies:
            c.wait()

        r2_v[...] = jnp.full((16,), BIG, jnp.float32)
        for j in range(K_NN):
            for w in range(SEG_W // 16):
                k = gath_v[pl.ds(j * SEG_W + w * 16, 16)]
                cnt = jnp.sum((k <= thr).astype(jnp.int32), axis=0)

                @pl.when(cnt > 0)
                def _merge():
                    kd, _ = plsc.sort_key_val(k, iota16, descending=True)
                    mk = jnp.minimum(r2_v[...], kd)
                    ks, _ = plsc.sort_key_val(mk, iota16)
                    r2_v[...] = ks

        x = jnp.maximum(r2_v[...], 1e-12)
        yi = 0x5F3759DF - lax.shift_right_logical(plsc.bitcast(x, jnp.int32),
                                                  1)
        y = plsc.bitcast(yi, jnp.float32)
        for _ in range(3):
            y = y * (1.5 - 0.5 * x * y * y)
        out_v[pl.ds(qi * 16, 16)] = x * y
        return 0

    lax.fori_loop(0, 16, one_query, 0)
    pltpu.sync_copy(out_v, out_hbm.at[pl.ds(q0 * 16, 256)])


@jax.jit
def kernel(patch, memory_bank):
    q_n, d = patch.shape
    n_rows = memory_bank.shape[0]
    n_tiles = pl.cdiv(n_rows, TILE)
    n_cols = n_tiles * TILE
    n_seg = n_cols // SEG_W
    n_qblk = q_n // QBLK

    dist, segmin = pl.pallas_call(
        functools.partial(_phase1_kernel, n_rows=n_rows, tile=TILE),
        grid=(n_tiles, n_qblk),
        in_specs=[
            pl.BlockSpec((QBLK, d), lambda i, j: (j, 0)),
            pl.BlockSpec((TILE, d), lambda i, j: (i, 0)),
        ],
        out_specs=[
            pl.BlockSpec((QBLK, TILE), lambda i, j: (j, i)),
            pl.BlockSpec((1, QBLK, TILE // SEG_W), lambda i, j: (i, j, 0)),
        ],
        out_shape=[
            jax.ShapeDtypeStruct((q_n, n_cols), jnp.float32),
            jax.ShapeDtypeStruct((n_tiles, q_n, TILE // SEG_W), jnp.float32),
        ],
        scratch_shapes=[pltpu.VMEM((1, TILE), jnp.float32)],
        compiler_params=pltpu.CompilerParams(
            dimension_semantics=("arbitrary", "arbitrary")),
    )(patch, memory_bank)

    segmin_flat = segmin.reshape(-1)

    mesh = plsc.VectorSubcoreMesh(core_axis_name="c", subcore_axis_name="s")
    out = pl.kernel(
        functools.partial(_phase2_kernel, n_seg=n_seg, n_q=q_n),
        mesh=mesh,
        out_type=jax.ShapeDtypeStruct((q_n * 16,), jnp.float32),
        scratch_types=[
            pltpu.VMEM((16 * n_seg,), jnp.float32),
            pltpu.VMEM((K_NN * SEG_W,), jnp.float32),
            pltpu.VMEM((16 * 16,), jnp.float32),
            pltpu.VMEM((16,), jnp.float32),
            pltpu.SemaphoreType.DMA,
        ],
        compiler_params=pltpu.CompilerParams(needs_layout_passes=False),
    )(dist, segmin_flat)
    return out.reshape(q_n, 16)[:, :K_NN]


# speedup vs baseline: 1.9812x; 1.3170x over previous
import functools

import jax
import jax.numpy as jnp
from jax import lax
from jax.experimental import pallas as pl
from jax.experimental.pallas import tpu as pltpu
from jax.experimental.pallas import tpu_sc as plsc

K_NN = 9
BIG = 1e30
SEG_W = 128
TILE = 2048
QBLK = 256


def _phase1_kernel(patch_ref, bank_ref, dist_ref, segmin_ref, b2_ref, *,
                   n_rows, tile):
    i = pl.program_id(0)
    j = pl.program_id(1)
    q = patch_ref[...]
    b = bank_ref[...]

    @pl.when(j == 0)
    def _b2():
        ones = jnp.ones((1, b.shape[1]), jnp.float32)
        b2_ref[...] = lax.dot_general(ones, b * b, (((1,), (1,)), ((), ())),
                                      preferred_element_type=jnp.float32)

    b2 = b2_ref[...]
    q2 = jnp.sum(q * q, axis=1, keepdims=True)
    dot = lax.dot_general(q, b, (((1,), (1,)), ((), ())),
                          preferred_element_type=jnp.float32)
    d2 = q2 + (b2 - 2.0 * dot)
    n_full = n_rows // tile

    @pl.when(i >= n_full)
    def _mask():
        col = lax.broadcasted_iota(jnp.int32, d2.shape, 1) + i * tile
        dist_ref[...] = jnp.where(col < n_rows, d2, BIG)

    @pl.when(i < n_full)
    def _store():
        dist_ref[...] = d2

    d = dist_ref[...]
    segs = [jnp.min(d[:, g * SEG_W:(g + 1) * SEG_W], axis=1, keepdims=True)
            for g in range(tile // SEG_W)]
    segmin_ref[...] = jnp.concatenate(segs, axis=1)[None]


def _phase2_kernel(dist_flat, segmin_flat, out_hbm, seg_v, gath_v, out_v,
                   r2_v, sem, *, n_seg, n_q):
    seg_c = n_seg // 16
    wid = lax.axis_index("s") * 2 + lax.axis_index("c")
    q0 = wid * 16
    iota16 = lax.iota(jnp.int32, 16)

    seg_copies = [
        pltpu.async_copy(
            segmin_flat.at[pl.ds((c * n_q + q0) * 16, 256)],
            seg_v.at[pl.ds(c * 256, 256)], sem)
        for c in range(seg_c)
    ]
    for cp in seg_copies:
        cp.wait()

    def one_query(qi, _):
        rk = seg_v[pl.ds(qi * 16, 16)]
        rv = iota16
        rk, rv = plsc.sort_key_val(rk, rv)

        def merge_seg(c, carry):
            rk, rv = carry
            k = seg_v[pl.ds(c * 256 + qi * 16, 16)]
            v = iota16 + c * 16
            kd, vd = plsc.sort_key_val(k, v, descending=True)
            mk = jnp.minimum(rk, kd)
            mv = jnp.where(kd < rk, vd, rv)
            return tuple(plsc.sort_key_val(mk, mv))

        rk, rv = lax.fori_loop(1, seg_c, merge_seg, (rk, rv))
        thr = jnp.max(jnp.where(iota16 == 8, rk, -BIG), axis=0)

        copies = []
        for j in range(K_NN):
            seg_j = jnp.max(jnp.where(iota16 == j, rv, -1), axis=0)
            copies.append(pltpu.async_copy(
                dist_flat.at[q0 + qi, pl.ds(seg_j * SEG_W, SEG_W)],
                gath_v.at[pl.ds(j * SEG_W, SEG_W)], sem))
        for c in copies:
            c.wait()

        r2_v[...] = jnp.full((16,), BIG, jnp.float32)
        for j in range(K_NN):
            for w in range(SEG_W // 16):
                k = gath_v[pl.ds(j * SEG_W + w * 16, 16)]
                cnt = jnp.sum((k <= thr).astype(jnp.int32), axis=0)

                @pl.when(cnt > 0)
                def _merge():
                    kd, _ = plsc.sort_key_val(k, iota16, descending=True)
                    mk = jnp.minimum(r2_v[...], kd)
                    ks, _ = plsc.sort_key_val(mk, iota16)
                    r2_v[...] = ks

        x = jnp.maximum(r2_v[...], 1e-12)
        yi = 0x5F3759DF - lax.shift_right_logical(plsc.bitcast(x, jnp.int32),
                                                  1)
        y = plsc.bitcast(yi, jnp.float32)
        for _ in range(3):
            y = y * (1.5 - 0.5 * x * y * y)
        out_v[pl.ds(qi * 16, 16)] = x * y
        return 0

    lax.fori_loop(0, 16, one_query, 0)
    pltpu.sync_copy(out_v, out_hbm.at[pl.ds(q0 * 16, 256)])


@jax.jit
def kernel(patch, memory_bank):
    q_n, d = patch.shape
    n_rows = memory_bank.shape[0]
    n_tiles = pl.cdiv(n_rows, TILE)
    n_cols = n_tiles * TILE
    n_seg = n_cols // SEG_W
    n_qblk = q_n // QBLK

    dist, segmin = pl.pallas_call(
        functools.partial(_phase1_kernel, n_rows=n_rows, tile=TILE),
        grid=(n_tiles, n_qblk),
        in_specs=[
            pl.BlockSpec((QBLK, d), lambda i, j: (j, 0)),
            pl.BlockSpec((TILE, d), lambda i, j: (i, 0)),
        ],
        out_specs=[
            pl.BlockSpec((QBLK, TILE), lambda i, j: (j, i)),
            pl.BlockSpec((1, QBLK, TILE // SEG_W), lambda i, j: (i, j, 0)),
        ],
        out_shape=[
            jax.ShapeDtypeStruct((q_n, n_cols), jnp.float32),
            jax.ShapeDtypeStruct((n_tiles, q_n, TILE // SEG_W), jnp.float32),
        ],
        scratch_shapes=[pltpu.VMEM((1, TILE), jnp.float32)],
        compiler_params=pltpu.CompilerParams(
            dimension_semantics=("arbitrary", "arbitrary")),
    )(patch, memory_bank)

    segmin_flat = segmin.reshape(-1)

    mesh = plsc.VectorSubcoreMesh(core_axis_name="c", subcore_axis_name="s")
    out = pl.kernel(
        functools.partial(_phase2_kernel, n_seg=n_seg, n_q=q_n),
        mesh=mesh,
        out_type=jax.ShapeDtypeStruct((q_n * 16,), jnp.float32),
        scratch_types=[
            pltpu.VMEM((16 * n_seg,), jnp.float32),
            pltpu.VMEM((K_NN * SEG_W,), jnp.float32),
            pltpu.VMEM((16 * 16,), jnp.float32),
            pltpu.VMEM((16,), jnp.float32),
            pltpu.SemaphoreType.DMA,
        ],
        compiler_params=pltpu.CompilerParams(needs_layout_passes=False),
    )(dist, segmin_flat)
    return out.reshape(q_n, 16)[:, :K_NN]


# speedup vs baseline: 2.3597x; 1.1910x over previous
import functools

import jax
import jax.numpy as jnp
from jax import lax
from jax.experimental import pallas as pl
from jax.experimental.pallas import tpu as pltpu
from jax.experimental.pallas import tpu_sc as plsc

K_NN = 9
BIG = 1e30
SEG_W = 256
TILE = 4096
QBLK = 256


def _phase1_kernel(patch_ref, bank_ref, dist_ref, segmin_ref, b2_ref, *,
                   n_rows, tile):
    i = pl.program_id(0)
    j = pl.program_id(1)
    q = patch_ref[...]
    b = bank_ref[...]

    @pl.when(j == 0)
    def _b2():
        ones = jnp.ones((1, b.shape[1]), jnp.float32)
        b2_ref[...] = lax.dot_general(ones, b * b, (((1,), (1,)), ((), ())),
                                      preferred_element_type=jnp.float32)

    b2 = b2_ref[...]
    q2 = jnp.sum(q * q, axis=1, keepdims=True)
    dot = lax.dot_general(q, b, (((1,), (1,)), ((), ())),
                          preferred_element_type=jnp.float32)
    d2 = q2 + (b2 - 2.0 * dot)
    n_full = n_rows // tile

    @pl.when(i >= n_full)
    def _mask():
        col = lax.broadcasted_iota(jnp.int32, d2.shape, 1) + i * tile
        dist_ref[...] = jnp.where(col < n_rows, d2, BIG)

    @pl.when(i < n_full)
    def _store():
        dist_ref[...] = d2

    d = dist_ref[...]
    segs = [jnp.min(d[:, g * SEG_W:(g + 1) * SEG_W], axis=1, keepdims=True)
            for g in range(tile // SEG_W)]
    segmin_ref[...] = jnp.concatenate(segs, axis=1)[None]


def _phase2_kernel(dist_flat, segmin_flat, out_hbm, seg_v, gath_v, out_v,
                   r2_v, sem, *, n_seg, n_q):
    seg_c = n_seg // 16
    wid = lax.axis_index("s") * 2 + lax.axis_index("c")
    q0 = wid * 16
    iota16 = lax.iota(jnp.int32, 16)

    seg_copies = [
        pltpu.async_copy(
            segmin_flat.at[pl.ds((c * n_q + q0) * 16, 256)],
            seg_v.at[pl.ds(c * 256, 256)], sem)
        for c in range(seg_c)
    ]
    for cp in seg_copies:
        cp.wait()

    def one_query(qi, _):
        rk = seg_v[pl.ds(qi * 16, 16)]
        rv = iota16
        rk, rv = plsc.sort_key_val(rk, rv)

        def merge_seg(c, carry):
            rk, rv = carry
            k = seg_v[pl.ds(c * 256 + qi * 16, 16)]
            v = iota16 + c * 16
            kd, vd = plsc.sort_key_val(k, v, descending=True)
            mk = jnp.minimum(rk, kd)
            mv = jnp.where(kd < rk, vd, rv)
            return tuple(plsc.sort_key_val(mk, mv))

        rk, rv = lax.fori_loop(1, seg_c, merge_seg, (rk, rv))
        thr = jnp.max(jnp.where(iota16 == 8, rk, -BIG), axis=0)

        copies = []
        for j in range(K_NN):
            seg_j = jnp.max(jnp.where(iota16 == j, rv, -1), axis=0)
            copies.append(pltpu.async_copy(
                dist_flat.at[q0 + qi, pl.ds(seg_j * SEG_W, SEG_W)],
                gath_v.at[pl.ds(j * SEG_W, SEG_W)], sem))
        for c in copies:
            c.wait()

        r2_v[...] = jnp.full((16,), BIG, jnp.float32)
        for j in range(K_NN):
            for w in range(SEG_W // 16):
                k = gath_v[pl.ds(j * SEG_W + w * 16, 16)]
                cnt = jnp.sum((k <= thr).astype(jnp.int32), axis=0)

                @pl.when(cnt > 0)
                def _merge():
                    kd, _ = plsc.sort_key_val(k, iota16, descending=True)
                    mk = jnp.minimum(r2_v[...], kd)
                    ks, _ = plsc.sort_key_val(mk, iota16)
                    r2_v[...] = ks

        x = jnp.maximum(r2_v[...], 1e-12)
        yi = 0x5F3759DF - lax.shift_right_logical(plsc.bitcast(x, jnp.int32),
                                                  1)
        y = plsc.bitcast(yi, jnp.float32)
        for _ in range(3):
            y = y * (1.5 - 0.5 * x * y * y)
        out_v[pl.ds(qi * 16, 16)] = x * y
        return 0

    lax.fori_loop(0, 16, one_query, 0)
    pltpu.sync_copy(out_v, out_hbm.at[pl.ds(q0 * 16, 256)])


@jax.jit
def kernel(patch, memory_bank):
    q_n, d = patch.shape
    n_rows = memory_bank.shape[0]
    n_tiles = pl.cdiv(n_rows, TILE)
    n_cols = n_tiles * TILE
    n_seg = n_cols // SEG_W
    n_qblk = q_n // QBLK

    dist, segmin = pl.pallas_call(
        functools.partial(_phase1_kernel, n_rows=n_rows, tile=TILE),
        grid=(n_tiles, n_qblk),
        in_specs=[
            pl.BlockSpec((QBLK, d), lambda i, j: (j, 0)),
            pl.BlockSpec((TILE, d), lambda i, j: (i, 0)),
        ],
        out_specs=[
            pl.BlockSpec((QBLK, TILE), lambda i, j: (j, i)),
            pl.BlockSpec((1, QBLK, TILE // SEG_W), lambda i, j: (i, j, 0)),
        ],
        out_shape=[
            jax.ShapeDtypeStruct((q_n, n_cols), jnp.float32),
            jax.ShapeDtypeStruct((n_tiles, q_n, TILE // SEG_W), jnp.float32),
        ],
        scratch_shapes=[pltpu.VMEM((1, TILE), jnp.float32)],
        compiler_params=pltpu.CompilerParams(
            dimension_semantics=("arbitrary", "arbitrary")),
    )(patch, memory_bank)

    segmin_flat = segmin.reshape(-1)

    mesh = plsc.VectorSubcoreMesh(core_axis_name="c", subcore_axis_name="s")
    out = pl.kernel(
        functools.partial(_phase2_kernel, n_seg=n_seg, n_q=q_n),
        mesh=mesh,
        out_type=jax.ShapeDtypeStruct((q_n * 16,), jnp.float32),
        scratch_types=[
            pltpu.VMEM((16 * n_seg,), jnp.float32),
            pltpu.VMEM((K_NN * SEG_W,), jnp.float32),
            pltpu.VMEM((16 * 16,), jnp.float32),
            pltpu.VMEM((16,), jnp.float32),
            pltpu.SemaphoreType.DMA,
        ],
        compiler_params=pltpu.CompilerParams(needs_layout_passes=False),
    )(dist, segmin_flat)
    return out.reshape(q_n, 16)[:, :K_NN]
